# baseline (speedup 1.0000x reference)
import jax
import jax.numpy as jnp
from jax import lax
from jax.experimental import pallas as pl
from jax.experimental.pallas import tpu as pltpu
from jax.experimental.pallas import tpu_sc as plsc

_SCORE_T = 0.05
_NMS_T = 0.5
_MAXDET = 100
_NEG = -1e30
_N = 5000
_C = 21
_NC = 2
_NS = 16
_L = 16
_ROWS = 320
_G = _ROWS // _L
_BIG = 1e9
_K = 4
_CAP = 12


def _nms_body(boxes_hbm, probs_hbm, obox_hbm, osc_hbm, olab_hbm,
              boxes_v, probs_v, soa_v, lab_v, pub_v, blk_v, ws_v,
              obox_v, osc_v, olab_v, shared):
    c = lax.axis_index("c")
    s = lax.axis_index("s")
    start = jnp.minimum(s * _ROWS, _N - _ROWS)
    start_f = start.astype(jnp.float32)
    iota = lax.iota(jnp.int32, _L)
    iota_f = iota.astype(jnp.float32)
    ones_f = jnp.full((_L,), 1.0, jnp.float32)
    ones_i = jnp.full((_L,), 1, jnp.int32)
    neg_v = jnp.full((_L,), _NEG, jnp.float32)
    neg2_v = jnp.full((_L,), 2.0 * _NEG, jnp.float32)

    pltpu.sync_copy(boxes_hbm.at[c].at[pl.ds(start, _ROWS), :], boxes_v)
    pltpu.sync_copy(probs_hbm.at[c].at[pl.ds(start, _ROWS), :], probs_v)

    def setup_g(g, _):
        rows = g * _L + iota
        m = plsc.load_gather(probs_v, [rows, jnp.zeros((_L,), jnp.int32)])
        labf = jnp.zeros((_L,), jnp.float32)
        for cls in range(1, _C):
            v = plsc.load_gather(probs_v, [rows, ones_i * cls])
            p = v > m
            m = jnp.where(p, v, m)
            labf = jnp.where(p, jnp.full((_L,), float(cls), jnp.float32), labf)
        valid = (m > _SCORE_T) & (labf != 0.0)
        sc = jnp.where(valid, m, neg_v)
        co = []
        for k in range(6):
            co.append(plsc.load_gather(boxes_v, [rows, ones_i * k]))
            soa_v[pl.ds(k * _ROWS + g * _L, _L)] = co[k]
        vol = (jnp.maximum(co[3] - co[0], 0.0) * jnp.maximum(co[4] - co[1], 0.0)
               * jnp.maximum(co[5] - co[2], 0.0))
        soa_v[pl.ds(6 * _ROWS + g * _L, _L)] = vol
        soa_v[pl.ds(7 * _ROWS + g * _L, _L)] = sc
        lab_v[pl.ds(g * _L, _L)] = labf
        return 0

    lax.fori_loop(0, _G, setup_g, 0)

    @pl.when(s == 0)
    def _prefill():
        for k in range(7):
            osc_v[pl.ds(k * _L, _L)] = -ones_f
            olab_v[pl.ds(k * _L, _L)] = -ones_f
        for k in range(40):
            obox_v[pl.ds(k * _L, _L)] = jnp.zeros((_L,), jnp.float32)

    def track_top2():
        m1 = neg2_v
        a1 = jnp.zeros((_L,), jnp.float32)
        m2 = neg2_v
        a2 = jnp.zeros((_L,), jnp.float32)
        for g in range(_G):
            scn = soa_v[pl.ds(7 * _ROWS + g * _L, _L)]
            idxv = g * _L + iota_f
            p1 = scn > m1
            p2 = (~p1) & (scn > m2)
            m2 = jnp.where(p1, m1, jnp.where(p2, scn, m2))
            a2 = jnp.where(p1, a1, jnp.where(p2, idxv, a2))
            m1 = jnp.where(p1, scn, m1)
            a1 = jnp.where(p1, idxv, a1)
        return m1, a1, m2, a2

    m1v0, a1v0, m2v0, a2v0 = track_top2()

    def outer_body(carry):
        r, count, cont, m1v, a1v, m2v, a2v = carry

        selm = iota < 0
        lis = []
        for j in range(_K):
            mj = jnp.max(jnp.where(selm, neg2_v, m1v))
            candm = (~selm) & (m1v == mj)
            aj = jnp.min(jnp.where(candm, a1v, _BIG))
            lane_j = plsc.all_reduce_ffs(candm & (a1v == aj))
            selm = selm | (iota == lane_j)
            lis.append(aj.astype(jnp.int32))
        bs = jnp.where(selm, m2v, m1v)
        bg = jnp.where(selm, a2v, a1v) + start_f
        bsx = jnp.max(bs)
        bgx = jnp.min(jnp.where(bs == bsx, bg, _BIG))
        for j in range(_K):
            lij = lis[j]
            row = plsc.load_gather(soa_v, [jnp.minimum(iota, 7) * _ROWS + lij])
            labg = plsc.load_gather(lab_v, [ones_i * lij])
            row = jnp.where(iota == 8, ones_f * (start + lij).astype(jnp.float32),
                            row)
            row = jnp.where(iota == 9, labg, row)
            if j == 0:
                row = jnp.where(iota == 10, ones_f * bsx, row)
                row = jnp.where(iota == 11, ones_f * bgx, row)
            pub_v[pl.ds(j * _L, _L)] = row

        slot = jnp.bitwise_and(r, 1)
        pltpu.sync_copy(
            pub_v, shared.at[pl.ds(slot * (_NS * _K * _L) + s * (_K * _L),
                                   _K * _L)])
        plsc.subcore_barrier()
        pltpu.sync_copy(shared.at[pl.ds(slot * (_NS * _K * _L), _NS * _K * _L)],
                        blk_v.at[pl.ds(0, _NS * _K * _L)])

        hs = plsc.load_gather(blk_v, [iota * (_K * _L) + 7])
        hg = plsc.load_gather(blk_v, [iota * (_K * _L) + 8])
        bsa = plsc.load_gather(blk_v, [iota * (_K * _L) + 10])
        bga = plsc.load_gather(blk_v, [iota * (_K * _L) + 11])
        bss = jnp.max(bsa)
        bgs = jnp.min(jnp.where(bsa == bss, bga, _BIG))
        pkv0 = hg * 8.0
        zf = jnp.zeros((_L,), jnp.float32)

        def scan_cond(cs):
            return cs[0]

        def scan_body(cs):
            go, hsv, pkv, m, cnt, ax1, ay1, az1, ax2, ay2, az2, avl, agd = cs
            gm = jnp.max(hsv)
            pmin = jnp.min(jnp.where(hsv == gm, pkv, _BIG))
            pmin_i = pmin.astype(jnp.int32)
            hp = jnp.bitwise_and(pmin_i, 7)
            cg = jnp.right_shift(pmin_i, 3)
            cg_f = cg.astype(jnp.float32)
            tst = plsc.all_reduce_ffs((hsv == gm) & (pkv == pmin))
            proceed = (gm > (_NEG / 2)) & (
                (gm > bss) | ((gm == bss) & (cg_f < bgs)))
            rbv = ones_i * (tst * (_K * _L) + hp * _L)
            crow = plsc.load_gather(blk_v, [rbv + iota])
            cb = [plsc.load_gather(blk_v, [rbv + k]) for k in range(7)]
            cgv = ones_f * cg_f
            lanelt = iota < m
            dupm = (agd == cgv) & lanelt
            dxs = jnp.maximum(jnp.minimum(cb[3], ax2) - jnp.maximum(cb[0], ax1),
                              0.0)
            dys = jnp.maximum(jnp.minimum(cb[4], ay2) - jnp.maximum(cb[1], ay1),
                              0.0)
            dzs = jnp.maximum(jnp.minimum(cb[5], az2) - jnp.maximum(cb[2], az1),
                              0.0)
            inter = dxs * dys * dzs
            union = cb[6] + avl - inter
            supm = (inter > _NMS_T * jnp.maximum(union, 1e-8)) & lanelt
            accept = proceed & (~jnp.any(dupm | supm))
            ins = (iota == m) & accept
            ax1 = jnp.where(ins, cb[0], ax1)
            ay1 = jnp.where(ins, cb[1], ay1)
            az1 = jnp.where(ins, cb[2], az1)
            ax2 = jnp.where(ins, cb[3], ax2)
            ay2 = jnp.where(ins, cb[4], ay2)
            az2 = jnp.where(ins, cb[5], az2)
            avl = jnp.where(ins, cb[6], avl)
            agd = jnp.where(ins, cgv, agd)

            recm = accept & (s == 0)
            plsc.store_scatter(osc_v, [ones_i * cnt], crow,
                               mask=(iota == 7) & recm)
            plsc.store_scatter(olab_v, [ones_i * cnt], crow,
                               mask=(iota == 9) & recm)
            plsc.store_scatter(obox_v, [cnt * 6 + iota], crow,
                               mask=(iota < 6) & recm)

            acc_i = accept.astype(jnp.int32)
            cnt = cnt + acc_i
            m = m + acc_i
            hp2 = hp + 1
            rb2 = ones_i * (tst * (_K * _L) + hp2 * _L)
            hs_n = plsc.load_gather(blk_v, [rb2 + 7])
            hg_n = plsc.load_gather(blk_v, [rb2 + 8])
            tl = (iota == tst) & proceed
            dead = hp2 >= _K
            hsv = jnp.where(tl, jnp.where(dead, neg2_v, hs_n), hsv)
            pkv = jnp.where(tl, hg_n * 8.0 + hp2.astype(jnp.float32), pkv)
            go = proceed & ~(accept & ((m >= _CAP) | (cnt >= _MAXDET)))
            return (go, hsv, pkv, m, cnt,
                    ax1, ay1, az1, ax2, ay2, az2, avl, agd)

        scan0 = (cont, hs, pkv0, jnp.int32(0), count,
                 zf, zf, zf, zf, zf, zf, zf, -ones_f)
        scan_out = lax.while_loop(scan_cond, scan_body, scan0)
        m_fin = scan_out[3]
        cnt_fin = scan_out[4]

        ws_v[pl.ds(0 * _L, _L)] = scan_out[5]
        ws_v[pl.ds(1 * _L, _L)] = scan_out[6]
        ws_v[pl.ds(2 * _L, _L)] = scan_out[7]
        ws_v[pl.ds(3 * _L, _L)] = scan_out[8]
        ws_v[pl.ds(4 * _L, _L)] = scan_out[9]
        ws_v[pl.ds(5 * _L, _L)] = scan_out[10]
        ws_v[pl.ds(6 * _L, _L)] = scan_out[11]
        ws_v[pl.ds(7 * _L, _L)] = scan_out[12]

        def sweep_pair(p, _):
            a0 = 2 * p
            a1 = jnp.minimum(2 * p + 1, m_fin - 1)

            def wf(k, a):
                return plsc.load_gather(ws_v, [ones_i * (k * _L + a)])

            w0 = [wf(k, a0) for k in range(6)]
            v0 = wf(6, a0)
            g0 = wf(7, a0)
            w1 = [wf(k, a1) for k in range(6)]
            v1 = wf(6, a1)
            g1 = wf(7, a1)
            for gd in (g0, g1):
                wloc = gd.astype(jnp.int32) - start
                in_rng = (wloc >= 0) & (wloc < _ROWS)
                wloc_c = jnp.minimum(jnp.maximum(wloc, 0), _ROWS - 1)
                plsc.store_scatter(soa_v, [ones_i * (7 * _ROWS) + wloc_c],
                                   neg_v, mask=(iota == 0) & in_rng)
            for g in range(_G):
                b0 = g * _L
                x1g = soa_v[pl.ds(0 * _ROWS + b0, _L)]
                y1g = soa_v[pl.ds(1 * _ROWS + b0, _L)]
                z1g = soa_v[pl.ds(2 * _ROWS + b0, _L)]
                x2g = soa_v[pl.ds(3 * _ROWS + b0, _L)]
                y2g = soa_v[pl.ds(4 * _ROWS + b0, _L)]
                z2g = soa_v[pl.ds(5 * _ROWS + b0, _L)]
                volg = soa_v[pl.ds(6 * _ROWS + b0, _L)]
                scg = soa_v[pl.ds(7 * _ROWS + b0, _L)]
                dx0 = jnp.maximum(jnp.minimum(w0[3], x2g)
                                  - jnp.maximum(w0[0], x1g), 0.0)
                dy0 = jnp.maximum(jnp.minimum(w0[4], y2g)
                                  - jnp.maximum(w0[1], y1g), 0.0)
                dz0 = jnp.maximum(jnp.minimum(w0[5], z2g)
                                  - jnp.maximum(w0[2], z1g), 0.0)
                i0 = dx0 * dy0 * dz0
                s0 = i0 > _NMS_T * jnp.maximum(v0 + volg - i0, 1e-8)
                dx1 = jnp.maximum(jnp.minimum(w1[3], x2g)
                                  - jnp.maximum(w1[0], x1g), 0.0)
                dy1 = jnp.maximum(jnp.minimum(w1[4], y2g)
                                  - jnp.maximum(w1[1], y1g), 0.0)
                dz1 = jnp.maximum(jnp.minimum(w1[5], z2g)
                                  - jnp.maximum(w1[2], z1g), 0.0)
                i1 = dx1 * dy1 * dz1
                s1 = i1 > _NMS_T * jnp.maximum(v1 + volg - i1, 1e-8)
                soa_v[pl.ds(7 * _ROWS + b0, _L)] = jnp.where(s0 | s1, neg_v,
                                                             scg)
            return 0

        lax.fori_loop(0, lax.div(m_fin + 1, 2), sweep_pair, 0)

        m1v, a1v, m2v, a2v = track_top2()
        cont2 = (cnt_fin < _MAXDET) & (m_fin > 0)
        return (r + 1, cnt_fin, cont2, m1v, a1v, m2v, a2v)

    lax.while_loop(lambda cy: cy[2], outer_body,
                   (jnp.int32(0), jnp.int32(0), jnp.bool_(True),
                    m1v0, a1v0, m2v0, a2v0))

    @pl.when(s == 0)
    def _writeback():
        pltpu.sync_copy(obox_v, obox_hbm.at[pl.ds(c * 640, 640)])
        pltpu.sync_copy(osc_v, osc_hbm.at[pl.ds(c * 112, 112)])
        pltpu.sync_copy(olab_v, olab_hbm.at[pl.ds(c * 112, 112)])


@jax.jit
def kernel(boxes, probs):
    mesh = plsc.VectorSubcoreMesh(core_axis_name="c", subcore_axis_name="s",
                                  num_cores=_NC, num_subcores=_NS)
    obox, osc, olab = pl.kernel(
        _nms_body,
        out_type=(
            jax.ShapeDtypeStruct((_NC * 640,), jnp.float32),
            jax.ShapeDtypeStruct((_NC * 112,), jnp.float32),
            jax.ShapeDtypeStruct((_NC * 112,), jnp.float32),
        ),
        mesh=mesh,
        compiler_params=pltpu.CompilerParams(needs_layout_passes=False, use_tc_tiling_on_sc=True),
        scratch_types=[
            pltpu.VMEM((_ROWS, 6), jnp.float32),
            pltpu.VMEM((_ROWS, _C), jnp.float32),
            pltpu.VMEM((8 * _ROWS,), jnp.float32),
            pltpu.VMEM((_ROWS,), jnp.float32),
            pltpu.VMEM((_K * _L,), jnp.float32),
            pltpu.VMEM((_NS * _K * _L + 5 * _L,), jnp.float32),
            pltpu.VMEM((8 * _L,), jnp.float32),
            pltpu.VMEM((640,), jnp.float32),
            pltpu.VMEM((112,), jnp.float32),
            pltpu.VMEM((112,), jnp.float32),
            pltpu.VMEM_SHARED((2 * _NS * _K * _L,), jnp.float32),
        ],
    )(boxes, probs)
    obox = obox.reshape(_NC, 640)
    osc = osc.reshape(_NC, 112)
    olab = olab.reshape(_NC, 112)
    boxes_out = obox[:, : 6 * _MAXDET].reshape(_NC, _MAXDET, 6)
    scores_out = osc[:, :_MAXDET]
    labels_out = olab[:, :_MAXDET].astype(jnp.int32)
    return boxes_out, scores_out, labels_out


# speedup vs baseline: 1.0038x; 1.0038x over previous
import jax
import jax.numpy as jnp
from jax import lax
from jax.experimental import pallas as pl
from jax.experimental.pallas import tpu as pltpu
from jax.experimental.pallas import tpu_sc as plsc

_SCORE_T = 0.05
_NMS_T = 0.5
_MAXDET = 100
_NEG = -1e30
_N = 5000
_C = 21
_NC = 2
_NS = 16
_L = 16
_ROWS = 320
_G = _ROWS // _L
_BIG = 1e9
_K = 6
_CAP = 16


def _nms_body(boxes_hbm, probs_hbm, obox_hbm, osc_hbm, olab_hbm,
              boxes_v, probs_v, soa_v, lab_v, pub_v, blk_v, ws_v,
              obox_v, osc_v, olab_v, shared):
    c = lax.axis_index("c")
    s = lax.axis_index("s")
    start = jnp.minimum(s * _ROWS, _N - _ROWS)
    start_f = start.astype(jnp.float32)
    iota = lax.iota(jnp.int32, _L)
    iota_f = iota.astype(jnp.float32)
    ones_f = jnp.full((_L,), 1.0, jnp.float32)
    ones_i = jnp.full((_L,), 1, jnp.int32)
    neg_v = jnp.full((_L,), _NEG, jnp.float32)
    neg2_v = jnp.full((_L,), 2.0 * _NEG, jnp.float32)

    pltpu.sync_copy(boxes_hbm.at[c].at[pl.ds(start, _ROWS), :], boxes_v)
    pltpu.sync_copy(probs_hbm.at[c].at[pl.ds(start, _ROWS), :], probs_v)

    def setup_g(g, _):
        rows = g * _L + iota
        m = plsc.load_gather(probs_v, [rows, jnp.zeros((_L,), jnp.int32)])
        labf = jnp.zeros((_L,), jnp.float32)
        for cls in range(1, _C):
            v = plsc.load_gather(probs_v, [rows, ones_i * cls])
            p = v > m
            m = jnp.where(p, v, m)
            labf = jnp.where(p, jnp.full((_L,), float(cls), jnp.float32), labf)
        valid = (m > _SCORE_T) & (labf != 0.0)
        sc = jnp.where(valid, m, neg_v)
        co = []
        for k in range(6):
            co.append(plsc.load_gather(boxes_v, [rows, ones_i * k]))
            soa_v[pl.ds(k * _ROWS + g * _L, _L)] = co[k]
        vol = (jnp.maximum(co[3] - co[0], 0.0) * jnp.maximum(co[4] - co[1], 0.0)
               * jnp.maximum(co[5] - co[2], 0.0))
        soa_v[pl.ds(6 * _ROWS + g * _L, _L)] = vol
        soa_v[pl.ds(7 * _ROWS + g * _L, _L)] = sc
        lab_v[pl.ds(g * _L, _L)] = labf
        return 0

    lax.fori_loop(0, _G, setup_g, 0)

    @pl.when(s == 0)
    def _prefill():
        for k in range(7):
            osc_v[pl.ds(k * _L, _L)] = -ones_f
            olab_v[pl.ds(k * _L, _L)] = -ones_f
        for k in range(40):
            obox_v[pl.ds(k * _L, _L)] = jnp.zeros((_L,), jnp.float32)

    def track_top2():
        m1 = neg2_v
        a1 = jnp.zeros((_L,), jnp.float32)
        m2 = neg2_v
        a2 = jnp.zeros((_L,), jnp.float32)
        for g in range(_G):
            scn = soa_v[pl.ds(7 * _ROWS + g * _L, _L)]
            idxv = g * _L + iota_f
            p1 = scn > m1
            p2 = (~p1) & (scn > m2)
            m2 = jnp.where(p1, m1, jnp.where(p2, scn, m2))
            a2 = jnp.where(p1, a1, jnp.where(p2, idxv, a2))
            m1 = jnp.where(p1, scn, m1)
            a1 = jnp.where(p1, idxv, a1)
        return m1, a1, m2, a2

    m1v0, a1v0, m2v0, a2v0 = track_top2()

    def outer_body(carry):
        r, count, cont, m1v, a1v, m2v, a2v = carry

        selm = iota < 0
        lis = []
        for j in range(_K):
            mj = jnp.max(jnp.where(selm, neg2_v, m1v))
            candm = (~selm) & (m1v == mj)
            aj = jnp.min(jnp.where(candm, a1v, _BIG))
            lane_j = plsc.all_reduce_ffs(candm & (a1v == aj))
            selm = selm | (iota == lane_j)
            lis.append(aj.astype(jnp.int32))
        bs = jnp.where(selm, m2v, m1v)
        bg = jnp.where(selm, a2v, a1v) + start_f
        bsx = jnp.max(bs)
        bgx = jnp.min(jnp.where(bs == bsx, bg, _BIG))
        for j in range(_K):
            lij = lis[j]
            row = plsc.load_gather(soa_v, [jnp.minimum(iota, 7) * _ROWS + lij])
            labg = plsc.load_gather(lab_v, [ones_i * lij])
            row = jnp.where(iota == 8, ones_f * (start + lij).astype(jnp.float32),
                            row)
            row = jnp.where(iota == 9, labg, row)
            if j == 0:
                row = jnp.where(iota == 10, ones_f * bsx, row)
                row = jnp.where(iota == 11, ones_f * bgx, row)
            pub_v[pl.ds(j * _L, _L)] = row

        slot = jnp.bitwise_and(r, 1)
        pltpu.sync_copy(
            pub_v, shared.at[pl.ds(slot * (_NS * _K * _L) + s * (_K * _L),
                                   _K * _L)])
        plsc.subcore_barrier()
        pltpu.sync_copy(shared.at[pl.ds(slot * (_NS * _K * _L), _NS * _K * _L)],
                        blk_v.at[pl.ds(0, _NS * _K * _L)])

        hs = plsc.load_gather(blk_v, [iota * (_K * _L) + 7])
        hg = plsc.load_gather(blk_v, [iota * (_K * _L) + 8])
        bsa = plsc.load_gather(blk_v, [iota * (_K * _L) + 10])
        bga = plsc.load_gather(blk_v, [iota * (_K * _L) + 11])
        bss = jnp.max(bsa)
        bgs = jnp.min(jnp.where(bsa == bss, bga, _BIG))
        pkv0 = hg * 8.0
        zf = jnp.zeros((_L,), jnp.float32)

        def scan_cond(cs):
            return cs[0]

        def scan_body(cs):
            go, hsv, pkv, m, cnt, ax1, ay1, az1, ax2, ay2, az2, avl, agd = cs
            gm = jnp.max(hsv)
            pmin = jnp.min(jnp.where(hsv == gm, pkv, _BIG))
            pmin_i = pmin.astype(jnp.int32)
            hp = jnp.bitwise_and(pmin_i, 7)
            cg = jnp.right_shift(pmin_i, 3)
            cg_f = cg.astype(jnp.float32)
            tst = plsc.all_reduce_ffs((hsv == gm) & (pkv == pmin))
            proceed = (gm > (_NEG / 2)) & (
                (gm > bss) | ((gm == bss) & (cg_f < bgs)))
            rbv = ones_i * (tst * (_K * _L) + hp * _L)
            crow = plsc.load_gather(blk_v, [rbv + iota])
            cb = [plsc.load_gather(blk_v, [rbv + k]) for k in range(7)]
            cgv = ones_f * cg_f
            lanelt = iota < m
            dupm = (agd == cgv) & lanelt
            dxs = jnp.maximum(jnp.minimum(cb[3], ax2) - jnp.maximum(cb[0], ax1),
                              0.0)
            dys = jnp.maximum(jnp.minimum(cb[4], ay2) - jnp.maximum(cb[1], ay1),
                              0.0)
            dzs = jnp.maximum(jnp.minimum(cb[5], az2) - jnp.maximum(cb[2], az1),
                              0.0)
            inter = dxs * dys * dzs
            union = cb[6] + avl - inter
            supm = (inter > _NMS_T * jnp.maximum(union, 1e-8)) & lanelt
            accept = proceed & (~jnp.any(dupm | supm))
            ins = (iota == m) & accept
            ax1 = jnp.where(ins, cb[0], ax1)
            ay1 = jnp.where(ins, cb[1], ay1)
            az1 = jnp.where(ins, cb[2], az1)
            ax2 = jnp.where(ins, cb[3], ax2)
            ay2 = jnp.where(ins, cb[4], ay2)
            az2 = jnp.where(ins, cb[5], az2)
            avl = jnp.where(ins, cb[6], avl)
            agd = jnp.where(ins, cgv, agd)

            recm = accept & (s == 0)
            plsc.store_scatter(osc_v, [ones_i * cnt], crow,
                               mask=(iota == 7) & recm)
            plsc.store_scatter(olab_v, [ones_i * cnt], crow,
                               mask=(iota == 9) & recm)
            plsc.store_scatter(obox_v, [cnt * 6 + iota], crow,
                               mask=(iota < 6) & recm)

            acc_i = accept.astype(jnp.int32)
            cnt = cnt + acc_i
            m = m + acc_i
            hp2 = hp + 1
            rb2 = ones_i * (tst * (_K * _L) + hp2 * _L)
            hs_n = plsc.load_gather(blk_v, [rb2 + 7])
            hg_n = plsc.load_gather(blk_v, [rb2 + 8])
            tl = (iota == tst) & proceed
            dead = hp2 >= _K
            hsv = jnp.where(tl, jnp.where(dead, neg2_v, hs_n), hsv)
            pkv = jnp.where(tl, hg_n * 8.0 + hp2.astype(jnp.float32), pkv)
            go = proceed & ~(accept & ((m >= _CAP) | (cnt >= _MAXDET)))
            return (go, hsv, pkv, m, cnt,
                    ax1, ay1, az1, ax2, ay2, az2, avl, agd)

        scan0 = (cont, hs, pkv0, jnp.int32(0), count,
                 zf, zf, zf, zf, zf, zf, zf, -ones_f)
        scan_out = lax.while_loop(scan_cond, scan_body, scan0)
        m_fin = scan_out[3]
        cnt_fin = scan_out[4]

        ws_v[pl.ds(0 * _L, _L)] = scan_out[5]
        ws_v[pl.ds(1 * _L, _L)] = scan_out[6]
        ws_v[pl.ds(2 * _L, _L)] = scan_out[7]
        ws_v[pl.ds(3 * _L, _L)] = scan_out[8]
        ws_v[pl.ds(4 * _L, _L)] = scan_out[9]
        ws_v[pl.ds(5 * _L, _L)] = scan_out[10]
        ws_v[pl.ds(6 * _L, _L)] = scan_out[11]
        ws_v[pl.ds(7 * _L, _L)] = scan_out[12]

        def sweep_pair(p, _):
            a0 = 2 * p
            a1 = jnp.minimum(2 * p + 1, m_fin - 1)

            def wf(k, a):
                return plsc.load_gather(ws_v, [ones_i * (k * _L + a)])

            w0 = [wf(k, a0) for k in range(6)]
            v0 = wf(6, a0)
            g0 = wf(7, a0)
            w1 = [wf(k, a1) for k in range(6)]
            v1 = wf(6, a1)
            g1 = wf(7, a1)
            for gd in (g0, g1):
                wloc = gd.astype(jnp.int32) - start
                in_rng = (wloc >= 0) & (wloc < _ROWS)
                wloc_c = jnp.minimum(jnp.maximum(wloc, 0), _ROWS - 1)
                plsc.store_scatter(soa_v, [ones_i * (7 * _ROWS) + wloc_c],
                                   neg_v, mask=(iota == 0) & in_rng)
            for g in range(_G):
                b0 = g * _L
                x1g = soa_v[pl.ds(0 * _ROWS + b0, _L)]
                y1g = soa_v[pl.ds(1 * _ROWS + b0, _L)]
                z1g = soa_v[pl.ds(2 * _ROWS + b0, _L)]
                x2g = soa_v[pl.ds(3 * _ROWS + b0, _L)]
                y2g = soa_v[pl.ds(4 * _ROWS + b0, _L)]
                z2g = soa_v[pl.ds(5 * _ROWS + b0, _L)]
                volg = soa_v[pl.ds(6 * _ROWS + b0, _L)]
                scg = soa_v[pl.ds(7 * _ROWS + b0, _L)]
                dx0 = jnp.maximum(jnp.minimum(w0[3], x2g)
                                  - jnp.maximum(w0[0], x1g), 0.0)
                dy0 = jnp.maximum(jnp.minimum(w0[4], y2g)
                                  - jnp.maximum(w0[1], y1g), 0.0)
                dz0 = jnp.maximum(jnp.minimum(w0[5], z2g)
                                  - jnp.maximum(w0[2], z1g), 0.0)
                i0 = dx0 * dy0 * dz0
                s0 = i0 > _NMS_T * jnp.maximum(v0 + volg - i0, 1e-8)
                dx1 = jnp.maximum(jnp.minimum(w1[3], x2g)
                                  - jnp.maximum(w1[0], x1g), 0.0)
                dy1 = jnp.maximum(jnp.minimum(w1[4], y2g)
                                  - jnp.maximum(w1[1], y1g), 0.0)
                dz1 = jnp.maximum(jnp.minimum(w1[5], z2g)
                                  - jnp.maximum(w1[2], z1g), 0.0)
                i1 = dx1 * dy1 * dz1
                s1 = i1 > _NMS_T * jnp.maximum(v1 + volg - i1, 1e-8)
                soa_v[pl.ds(7 * _ROWS + b0, _L)] = jnp.where(s0 | s1, neg_v,
                                                             scg)
            return 0

        lax.fori_loop(0, lax.div(m_fin + 1, 2), sweep_pair, 0)

        m1v, a1v, m2v, a2v = track_top2()
        cont2 = (cnt_fin < _MAXDET) & (m_fin > 0)
        return (r + 1, cnt_fin, cont2, m1v, a1v, m2v, a2v)

    lax.while_loop(lambda cy: cy[2], outer_body,
                   (jnp.int32(0), jnp.int32(0), jnp.bool_(True),
                    m1v0, a1v0, m2v0, a2v0))

    @pl.when(s == 0)
    def _writeback():
        pltpu.sync_copy(obox_v, obox_hbm.at[pl.ds(c * 640, 640)])
        pltpu.sync_copy(osc_v, osc_hbm.at[pl.ds(c * 112, 112)])
        pltpu.sync_copy(olab_v, olab_hbm.at[pl.ds(c * 112, 112)])


@jax.jit
def kernel(boxes, probs):
    mesh = plsc.VectorSubcoreMesh(core_axis_name="c", subcore_axis_name="s",
                                  num_cores=_NC, num_subcores=_NS)
    obox, osc, olab = pl.kernel(
        _nms_body,
        out_type=(
            jax.ShapeDtypeStruct((_NC * 640,), jnp.float32),
            jax.ShapeDtypeStruct((_NC * 112,), jnp.float32),
            jax.ShapeDtypeStruct((_NC * 112,), jnp.float32),
        ),
        mesh=mesh,
        compiler_params=pltpu.CompilerParams(needs_layout_passes=False),
        scratch_types=[
            pltpu.VMEM((_ROWS, 6), jnp.float32),
            pltpu.VMEM((_ROWS, _C), jnp.float32),
            pltpu.VMEM((8 * _ROWS,), jnp.float32),
            pltpu.VMEM((_ROWS,), jnp.float32),
            pltpu.VMEM((_K * _L,), jnp.float32),
            pltpu.VMEM((_NS * _K * _L + 5 * _L,), jnp.float32),
            pltpu.VMEM((8 * _L,), jnp.float32),
            pltpu.VMEM((640,), jnp.float32),
            pltpu.VMEM((112,), jnp.float32),
            pltpu.VMEM((112,), jnp.float32),
            pltpu.VMEM_SHARED((2 * _NS * _K * _L,), jnp.float32),
        ],
    )(boxes, probs)
    obox = obox.reshape(_NC, 640)
    osc = osc.reshape(_NC, 112)
    olab = olab.reshape(_NC, 112)
    boxes_out = obox[:, : 6 * _MAXDET].reshape(_NC, _MAXDET, 6)
    scores_out = osc[:, :_MAXDET]
    labels_out = olab[:, :_MAXDET].astype(jnp.int32)
    return boxes_out, scores_out, labels_out
